# Initial kernel scaffold; baseline (speedup 1.0000x reference)
import functools

import jax
import jax.numpy as jnp
from jax import lax
from jax.experimental import pallas as pl
from jax.experimental.pallas import tpu as pltpu
from jax.experimental.pallas import tpu_sc as plsc

N = 10000
E = 320000
D = 128
VC = 16
V3 = 3 * VC

NC = 2
NS = 16
NW = NC * NS
EPW = E // NW
CHUNK = 200
NCHUNK = EPW // CHUNK

BE = 2000
BN = 1000


def _edge_body(rad_ref, vec_ref, wr1_ref, wr2_ref, ew_ref, v48_ref):
    h = jax.nn.silu(jnp.dot(rad_ref[...], wr1_ref[...],
                            preferred_element_type=jnp.float32))
    ew = jnp.dot(h, wr2_ref[...], preferred_element_type=jnp.float32)
    ew_ref[...] = ew
    v = vec_ref[...]
    r = jnp.sqrt(v[:, 0:1] ** 2 + v[:, 1:2] ** 2 + v[:, 2:3] ** 2)
    rinv = 1.0 / (r + 1e-9)
    ew16 = ew[:, :VC]
    v48_ref[...] = jnp.concatenate(
        [(v[:, i:i + 1] * rinv) * ew16 for i in range(3)], axis=1)


_edge_call = pl.pallas_call(
    _edge_body,
    grid=(E // BE,),
    in_specs=[
        pl.BlockSpec((BE, 8), lambda i: (i, 0)),
        pl.BlockSpec((BE, 3), lambda i: (i, 0)),
        pl.BlockSpec((8, 64), lambda i: (0, 0)),
        pl.BlockSpec((64, D), lambda i: (0, 0)),
    ],
    out_specs=[
        pl.BlockSpec((BE, D), lambda i: (i, 0)),
        pl.BlockSpec((BE, V3), lambda i: (i, 0)),
    ],
    out_shape=[
        jax.ShapeDtypeStruct((E, D), jnp.float32),
        jax.ShapeDtypeStruct((E, V3), jnp.float32),
    ],
)


_sc_mesh = plsc.VectorSubcoreMesh(core_axis_name="c", subcore_axis_name="s")


@functools.partial(
    pl.kernel,
    out_type=(
        jax.ShapeDtypeStruct((NC, N, D), jnp.float32),
        jax.ShapeDtypeStruct((NC, N, V3), jnp.float32),
    ),
    mesh=_sc_mesh,
    scratch_types=[
        pltpu.VMEM((CHUNK,), jnp.int32),
        pltpu.VMEM((CHUNK,), jnp.int32),
        pltpu.VMEM((CHUNK, D), jnp.float32),
        pltpu.VMEM((CHUNK, D), jnp.float32),
        pltpu.VMEM((CHUNK, V3), jnp.float32),
        pltpu.VMEM_SHARED((N, D), jnp.float32),
        pltpu.VMEM_SHARED((N, V3), jnp.float32),
        pltpu.SemaphoreType.DMA,
    ],
)
def _sc_scatter(nf_hbm, send_hbm, recv_hbm, ew_hbm, v48_hbm, zA_hbm, zB_hbm,
                outA_hbm, outB_hbm,
                sidx, ridx, gath, ewv, v48v, accA, accB, sem):
    c = lax.axis_index("c")
    s = lax.axis_index("s")
    wid = s * NC + c

    @pl.when(s == 0)
    def _zero():
        pltpu.sync_copy(zA_hbm, accA)
        pltpu.sync_copy(zB_hbm, accB)

    plsc.subcore_barrier()

    def chunk(t, carry):
        base = pl.multiple_of(wid * EPW + t * CHUNK, 8)
        pltpu.sync_copy(send_hbm.at[pl.ds(base, CHUNK)], sidx)
        pltpu.sync_copy(recv_hbm.at[pl.ds(base, CHUNK)], ridx)
        pltpu.async_copy(nf_hbm.at[sidx], gath, sem).wait()
        pltpu.sync_copy(ew_hbm.at[pl.ds(base, CHUNK)], ewv)
        pltpu.sync_copy(v48_hbm.at[pl.ds(base, CHUNK)], v48v)

        def mul(i, carry2):
            for k in range(D // 16):
                sl = pl.ds(k * 16, 16)
                ewv[i, sl] = ewv[i, sl] * gath[i, sl]
            return carry2

        lax.fori_loop(0, CHUNK, mul, 0)
        pltpu.sync_copy(ewv, accA.at[ridx], add=True)
        pltpu.sync_copy(v48v, accB.at[ridx], add=True)
        return carry

    lax.fori_loop(0, NCHUNK, chunk, 0)
    plsc.subcore_barrier()

    rows = N // NS
    off = s * rows
    pltpu.sync_copy(accA.at[pl.ds(off, rows)],
                    outA_hbm.at[c, pl.ds(off, rows)])
    pltpu.sync_copy(accB.at[pl.ds(off, rows)],
                    outB_hbm.at[c, pl.ds(off, rows)])


def _node_body(accA_ref, accB_ref, nf_ref, ns_ref, sep_ref, wmsg_ref, wvec_ref,
               wsc_ref, wres_ref, lns_ref, wro_ref, x_ref, ro_ref):
    agg = accA_ref[0] + accA_ref[1]
    vb = accB_ref[0] + accB_ref[1]
    vn = jnp.sqrt(vb[:, 0:VC] ** 2 + vb[:, VC:2 * VC] ** 2
                  + vb[:, 2 * VC:3 * VC] ** 2 + 1e-9)
    x = (jnp.dot(agg, wmsg_ref[...], preferred_element_type=jnp.float32)
         + jnp.dot(vn, wvec_ref[...], preferred_element_type=jnp.float32))
    oh = (ns_ref[...] == lax.broadcasted_iota(jnp.int32, (BN, 16), 1)
          ).astype(jnp.float32)
    sp = jnp.dot(oh, sep_ref[...], preferred_element_type=jnp.float32)
    x = x + jnp.tanh(jnp.dot(sp, wsc_ref[...],
                             preferred_element_type=jnp.float32))
    mu = jnp.mean(x, axis=1, keepdims=True)
    xc = x - mu
    var = jnp.mean(xc * xc, axis=1, keepdims=True)
    xln = xc * lax.rsqrt(var + 1e-6) * lns_ref[...]
    x = xln + jnp.dot(nf_ref[...], wres_ref[...],
                      preferred_element_type=jnp.float32)
    x_ref[...] = x
    ro_ref[...] = jnp.dot(x, wro_ref[...], preferred_element_type=jnp.float32)


_node_call = pl.pallas_call(
    _node_body,
    grid=(N // BN,),
    in_specs=[
        pl.BlockSpec((NC, BN, D), lambda i: (0, i, 0)),
        pl.BlockSpec((NC, BN, V3), lambda i: (0, i, 0)),
        pl.BlockSpec((BN, D), lambda i: (i, 0)),
        pl.BlockSpec((BN, 1), lambda i: (i, 0)),
        pl.BlockSpec((16, 64), lambda i: (0, 0)),
        pl.BlockSpec((D, D), lambda i: (0, 0)),
        pl.BlockSpec((VC, D), lambda i: (0, 0)),
        pl.BlockSpec((64, D), lambda i: (0, 0)),
        pl.BlockSpec((D, D), lambda i: (0, 0)),
        pl.BlockSpec((1, D), lambda i: (0, 0)),
        pl.BlockSpec((D, 1), lambda i: (0, 0)),
    ],
    out_specs=[
        pl.BlockSpec((BN, D), lambda i: (i, 0)),
        pl.BlockSpec((BN, 1), lambda i: (i, 0)),
    ],
    out_shape=[
        jax.ShapeDtypeStruct((N, D), jnp.float32),
        jax.ShapeDtypeStruct((N, 1), jnp.float32),
    ],
)


def kernel(vectors, node_feats, node_species, radial_embedding, receivers,
           senders, species_embed, W_r1, W_r2, W_vec, W_msg, W_sc, W_resid,
           ln_scale, W_readout):
    ew, v48 = _edge_call(radial_embedding, vectors, W_r1, W_r2)
    zA = jnp.zeros((N, D), jnp.float32)
    zB = jnp.zeros((N, V3), jnp.float32)
    outA, outB = _sc_scatter(node_feats, senders.astype(jnp.int32),
                             receivers.astype(jnp.int32), ew, v48, zA, zB)
    sep = jnp.zeros((16, 64), jnp.float32).at[:10, :].set(species_embed)
    x, ro = _node_call(outA, outB, node_feats,
                       node_species.reshape(N, 1).astype(jnp.int32), sep,
                       W_msg, W_vec, W_sc, W_resid,
                       ln_scale.reshape(1, D), W_readout)
    return (x, ro)


# speedup vs baseline: 17.1254x; 17.1254x over previous
import functools

import jax
import jax.numpy as jnp
from jax import lax
from jax.experimental import pallas as pl
from jax.experimental.pallas import tpu as pltpu
from jax.experimental.pallas import tpu_sc as plsc

N = 10000
E = 320000
D = 128
VC = 16

NC = 2
NS = 16
DH = D // NC
VH = 3 * (VC // NC)
PC = DH + VH
EPS = E // NS
CHUNK = 80
NCHUNK = EPS // CHUNK

BE = 2000
BN = 1000


def _edge_body(rad_ref, vec_ref, wr1_ref, wr2_ref, p_ref):
    h = jax.nn.silu(jnp.dot(rad_ref[...], wr1_ref[...],
                            preferred_element_type=jnp.float32))
    ew = jnp.dot(h, wr2_ref[...], preferred_element_type=jnp.float32)
    v = vec_ref[...]
    r = jnp.sqrt(v[:, 0:1] ** 2 + v[:, 1:2] ** 2 + v[:, 2:3] ** 2)
    rinv = 1.0 / (r + 1e-9)
    vh = [v[:, i:i + 1] * rinv for i in range(3)]
    for cc in range(NC):
        ew8 = ew[:, cc * (VC // NC):(cc + 1) * (VC // NC)]
        p_ref[cc] = jnp.concatenate(
            [ew[:, cc * DH:(cc + 1) * DH]] + [vhi * ew8 for vhi in vh], axis=1)


_edge_call = pl.pallas_call(
    _edge_body,
    grid=(E // BE,),
    in_specs=[
        pl.BlockSpec((BE, 8), lambda i: (i, 0)),
        pl.BlockSpec((BE, 3), lambda i: (i, 0)),
        pl.BlockSpec((8, 64), lambda i: (0, 0)),
        pl.BlockSpec((64, D), lambda i: (0, 0)),
    ],
    out_specs=pl.BlockSpec((NC, BE, PC), lambda i: (0, i, 0)),
    out_shape=jax.ShapeDtypeStruct((NC, E, PC), jnp.float32),
)


def _sc_body(nf_hbm, send_hbm, recv_hbm, p_hbm, out_hbm,
             sidx, ridx, gath, pv, acc, sem):
    c = lax.axis_index("c")
    s = lax.axis_index("s")

    def zrow(i, carry):
        for k in range(PC // 8):
            pv[i, pl.ds(k * 8, 8)] = jnp.zeros((8,), jnp.float32)
        return carry

    lax.fori_loop(0, CHUNK, zrow, 0)
    nslab = N // CHUNK

    def zslab(m, carry):
        slab = s + m * NS

        @pl.when(slab < nslab)
        def _():
            pltpu.sync_copy(pv, acc.at[pl.ds(slab * CHUNK, CHUNK)])

        return carry

    lax.fori_loop(0, (nslab + NS - 1) // NS, zslab, 0)

    plsc.subcore_barrier()

    def chunk(t, carry):
        base = pl.multiple_of(s * EPS + t * CHUNK, 8)
        pltpu.sync_copy(send_hbm.at[pl.ds(base, CHUNK)], sidx)
        pltpu.sync_copy(recv_hbm.at[pl.ds(base, CHUNK)], ridx)
        pltpu.async_copy(nf_hbm.at[sidx], gath, sem).wait()
        pltpu.sync_copy(p_hbm.at[c].at[pl.ds(base, CHUNK)], pv)

        coff = c * DH

        def mul(i, carry2):
            for k in range(DH // 16):
                pv[i, pl.ds(k * 16, 16)] = (pv[i, pl.ds(k * 16, 16)]
                                            * gath[i, pl.ds(coff + k * 16, 16)])
            return carry2

        lax.fori_loop(0, CHUNK, mul, 0)
        pltpu.sync_copy(pv, acc.at[ridx], add=True)
        return carry

    lax.fori_loop(0, NCHUNK, chunk, 0)
    plsc.subcore_barrier()

    @pl.when(s == 0)
    def _dump():
        pltpu.sync_copy(acc, out_hbm.at[c])


@functools.lru_cache(maxsize=1)
def _sc_call():
    mesh = plsc.VectorSubcoreMesh(core_axis_name="c", subcore_axis_name="s",
                                  num_cores=NC, num_subcores=NS)
    return pl.kernel(
        _sc_body,
        out_type=jax.ShapeDtypeStruct((NC, N, PC), jnp.float32),
        mesh=mesh,
        scratch_types=[
            pltpu.VMEM((CHUNK,), jnp.int32),
            pltpu.VMEM((CHUNK,), jnp.int32),
            pltpu.VMEM((CHUNK, D), jnp.float32),
            pltpu.VMEM((CHUNK, PC), jnp.float32),
            pltpu.VMEM_SHARED((N, PC), jnp.float32),
            pltpu.SemaphoreType.DMA,
        ],
    )


def _node_body(p_ref, nf_ref, ns_ref, sep_ref, wmsg_ref, wvec_ref,
               wsc_ref, wres_ref, lns_ref, wro_ref, x_ref, ro_ref):
    agg = jnp.concatenate([p_ref[0, :, :DH], p_ref[1, :, :DH]], axis=1)
    HV = VC // NC
    vns = []
    for cc in range(NC):
        vb = p_ref[cc, :, DH:]
        vns.append(jnp.sqrt(vb[:, 0:HV] ** 2 + vb[:, HV:2 * HV] ** 2
                            + vb[:, 2 * HV:3 * HV] ** 2 + 1e-9))
    vn = jnp.concatenate(vns, axis=1)
    x = (jnp.dot(agg, wmsg_ref[...], preferred_element_type=jnp.float32)
         + jnp.dot(vn, wvec_ref[...], preferred_element_type=jnp.float32))
    oh = (ns_ref[...] == lax.broadcasted_iota(jnp.int32, (BN, 16), 1)
          ).astype(jnp.float32)
    sp = jnp.dot(oh, sep_ref[...], preferred_element_type=jnp.float32)
    x = x + jnp.tanh(jnp.dot(sp, wsc_ref[...],
                             preferred_element_type=jnp.float32))
    mu = jnp.mean(x, axis=1, keepdims=True)
    xc = x - mu
    var = jnp.mean(xc * xc, axis=1, keepdims=True)
    xln = xc * lax.rsqrt(var + 1e-6) * lns_ref[...]
    x = xln + jnp.dot(nf_ref[...], wres_ref[...],
                      preferred_element_type=jnp.float32)
    x_ref[...] = x
    ro_ref[...] = jnp.dot(x, wro_ref[...], preferred_element_type=jnp.float32)


_node_call = pl.pallas_call(
    _node_body,
    grid=(N // BN,),
    in_specs=[
        pl.BlockSpec((NC, BN, PC), lambda i: (0, i, 0)),
        pl.BlockSpec((BN, D), lambda i: (i, 0)),
        pl.BlockSpec((BN, 1), lambda i: (i, 0)),
        pl.BlockSpec((16, 64), lambda i: (0, 0)),
        pl.BlockSpec((D, D), lambda i: (0, 0)),
        pl.BlockSpec((VC, D), lambda i: (0, 0)),
        pl.BlockSpec((64, D), lambda i: (0, 0)),
        pl.BlockSpec((D, D), lambda i: (0, 0)),
        pl.BlockSpec((1, D), lambda i: (0, 0)),
        pl.BlockSpec((D, 1), lambda i: (0, 0)),
    ],
    out_specs=[
        pl.BlockSpec((BN, D), lambda i: (i, 0)),
        pl.BlockSpec((BN, 1), lambda i: (i, 0)),
    ],
    out_shape=[
        jax.ShapeDtypeStruct((N, D), jnp.float32),
        jax.ShapeDtypeStruct((N, 1), jnp.float32),
    ],
)


def kernel(vectors, node_feats, node_species, radial_embedding, receivers,
           senders, species_embed, W_r1, W_r2, W_vec, W_msg, W_sc, W_resid,
           ln_scale, W_readout):
    payload = _edge_call(radial_embedding, vectors, W_r1, W_r2)
    out = _sc_call()(node_feats, senders.astype(jnp.int32),
                     receivers.astype(jnp.int32), payload)
    sep = jnp.zeros((16, 64), jnp.float32).at[:10, :].set(species_embed)
    x, ro = _node_call(out, node_feats,
                       node_species.reshape(N, 1).astype(jnp.int32), sep,
                       W_msg, W_vec, W_sc, W_resid,
                       ln_scale.reshape(1, D), W_readout)
    return (x, ro)
